# baseline (speedup 1.0000x reference)
import functools

import jax
import jax.numpy as jnp
from jax import lax
from jax.experimental import pallas as pl
from jax.experimental.pallas import tpu as pltpu
from jax.experimental.pallas import tpu_sc as plsc

_N = 320000
_D = 128
_C = 1000
_CPAD = 1024
_MOM = 0.1

_NC = 2
_NS = 16
_NW = _NC * _NS
_ROWS_PER = _N // _NW
_CH = 200
_NITER = _ROWS_PER // _CH
_NPAIR = _NITER // 2
_RPT = _CPAD // _NS


def _sc_segment_sums(z, y, zsum, ones):
    mesh = plsc.VectorSubcoreMesh(
        core_axis_name="c", subcore_axis_name="s",
        num_cores=_NC, num_subcores=_NS)

    @functools.partial(
        pl.kernel,
        out_type=(
            jax.ShapeDtypeStruct((_NC, _CPAD, _D), jnp.float32),
            jax.ShapeDtypeStruct((_NC, _CPAD, _D), jnp.float32),
        ),
        mesh=mesh,
        scratch_types=[
            pltpu.VMEM((_CH, _D), jnp.float32),
            pltpu.VMEM((_CH, _D), jnp.float32),
            pltpu.VMEM((_CH,), jnp.int32),
            pltpu.VMEM((_CH,), jnp.int32),
            pltpu.VMEM((_CH, _D), jnp.float32),
            pltpu.VMEM_SHARED((_CPAD, _D), jnp.float32),
            pltpu.VMEM_SHARED((_CPAD, _D), jnp.float32),
            pltpu.SemaphoreType.DMA,
            pltpu.SemaphoreType.DMA,
            pltpu.SemaphoreType.DMA,
            pltpu.SemaphoreType.DMA,
        ],
    )
    def k(z_hbm, y_hbm, zsum_hbm, ones_hbm, osum_hbm, ocnt_hbm,
          zbuf0, zbuf1, idx0, idx1, onesbuf, acc_sh, cnt_sh,
          sem0, sem1, ssem0, ssem1):
        c = lax.axis_index("c")
        s = lax.axis_index("s")
        wid = c * _NS + s
        base = wid * _ROWS_PER
        zbufs, idxs = (zbuf0, zbuf1), (idx0, idx1)
        sems, ssems = (sem0, sem1), (ssem0, ssem1)
        pltpu.sync_copy(zsum_hbm.at[pl.ds(s * _RPT, _RPT)],
                        acc_sh.at[pl.ds(s * _RPT, _RPT)])
        pltpu.sync_copy(zsum_hbm.at[pl.ds(s * _RPT, _RPT)],
                        cnt_sh.at[pl.ds(s * _RPT, _RPT)])
        pltpu.sync_copy(ones_hbm, onesbuf)
        plsc.subcore_barrier()

        def start_load(chunk, slot):
            off = base + chunk * _CH
            pltpu.async_copy(z_hbm.at[pl.ds(off, _CH)], zbufs[slot], sems[slot])
            pltpu.async_copy(y_hbm.at[pl.ds(off, _CH)], idxs[slot], sems[slot])

        def wait_load(chunk, slot):
            off = base + chunk * _CH
            pltpu.make_async_copy(z_hbm.at[pl.ds(off, _CH)], zbufs[slot],
                                  sems[slot]).wait()
            pltpu.make_async_copy(y_hbm.at[pl.ds(off, _CH)], idxs[slot],
                                  sems[slot]).wait()

        def start_scatter(slot):
            pltpu.async_copy(zbufs[slot], acc_sh.at[idxs[slot]], ssems[slot],
                             add=True)
            pltpu.async_copy(onesbuf, cnt_sh.at[idxs[slot]], ssems[slot],
                             add=True)

        def wait_scatter(slot):
            pltpu.make_async_copy(zbufs[slot], acc_sh.at[idxs[slot]],
                                  ssems[slot]).wait()
            pltpu.make_async_copy(onesbuf, cnt_sh.at[idxs[slot]],
                                  ssems[slot]).wait()

        start_load(0, 0)
        start_load(1, 1)

        def body(g, carry):
            i0 = g * 2
            wait_load(i0, 0)
            start_scatter(0)
            wait_load(i0 + 1, 1)
            start_scatter(1)
            nxt0 = jnp.where(i0 + 2 < _NITER, i0 + 2, 0)
            nxt1 = jnp.where(i0 + 3 < _NITER, i0 + 3, 0)
            wait_scatter(0)
            start_load(nxt0, 0)
            wait_scatter(1)
            start_load(nxt1, 1)
            return carry

        lax.fori_loop(0, _NPAIR, body, 0)
        wait_load(0, 0)
        wait_load(0, 1)
        plsc.subcore_barrier()
        pltpu.sync_copy(acc_sh.at[pl.ds(s * _RPT, _RPT)],
                        osum_hbm.at[c, pl.ds(s * _RPT, _RPT)])
        pltpu.sync_copy(cnt_sh.at[pl.ds(s * _RPT, _RPT)],
                        ocnt_hbm.at[c, pl.ds(s * _RPT, _RPT)])

    return k(z, y, zsum, ones)


def _combine_body(psum_ref, pcnt_ref, cen_ref, init_ref, oc_ref, oi_ref):
    sums = (psum_ref[0] + psum_ref[1])[:_C]
    counts = (pcnt_ref[0] + pcnt_ref[1])[:_C, 0:1]
    centers = cen_ref[...]
    init = init_ref[...]
    present = counts > 0.0
    safe = jnp.where(present, counts, 1.0)
    means = sums / safe
    ema = (1.0 - _MOM) * centers + _MOM * means
    upd = jnp.where(init > 0.0, ema, means)
    oc_ref[...] = jnp.where(present, upd, centers)
    oi_ref[...] = jnp.where(present, 1.0, init)


def kernel(z, y, centers, initialized):
    y = y.astype(jnp.int32)
    zsum = jnp.zeros((_CPAD, _D), jnp.float32)
    ones = jnp.ones((_CH, _D), jnp.float32)
    psum, pcnt = _sc_segment_sums(z, y, zsum, ones)

    init_f = initialized.astype(jnp.float32).reshape(_C, 1)
    new_centers, new_init = pl.pallas_call(
        _combine_body,
        out_shape=(
            jax.ShapeDtypeStruct((_C, _D), jnp.float32),
            jax.ShapeDtypeStruct((_C, 1), jnp.float32),
        ),
    )(psum, pcnt, centers, init_f)
    return new_centers, new_init.reshape(_C) > 0.0


# speedup vs baseline: 1.6953x; 1.6953x over previous
import functools

import jax
import jax.numpy as jnp
from jax import lax
from jax.experimental import pallas as pl
from jax.experimental.pallas import tpu as pltpu
from jax.experimental.pallas import tpu_sc as plsc

_N = 320000
_D = 128
_C = 1000
_CPAD = 1024
_MOM = 0.1

_NC = 2
_NS = 16
_NW = _NC * _NS
_ROWS_PER = _N // _NW
_CH = 200
_NITER = _ROWS_PER // _CH
_NPAIR = _NITER // 2
_RPT = _CPAD // _NS

_HROWS = 2560
_HP = _HROWS * _D
_HSUB = 128


def _sc_segment_sums(z, y, zsum):
    mesh = plsc.VectorSubcoreMesh(
        core_axis_name="c", subcore_axis_name="s",
        num_cores=_NC, num_subcores=_NS)

    @functools.partial(
        pl.kernel,
        out_type=jax.ShapeDtypeStruct((_NC, _CPAD, _D), jnp.float32),
        mesh=mesh,
        scratch_types=[
            pltpu.VMEM((_CH, _D), jnp.float32),
            pltpu.VMEM((_CH, _D), jnp.float32),
            pltpu.VMEM((_CH,), jnp.int32),
            pltpu.VMEM((_CH,), jnp.int32),
            pltpu.VMEM_SHARED((_CPAD, _D), jnp.float32),
            pltpu.SemaphoreType.DMA,
            pltpu.SemaphoreType.DMA,
        ],
    )
    def k(z_hbm, y_hbm, zsum_hbm, osum_hbm,
          zbuf0, zbuf1, idx0, idx1, acc_sh, sem0, sem1):
        c = lax.axis_index("c")
        s = lax.axis_index("s")
        wid = c * _NS + s
        base = wid * _ROWS_PER
        zbufs, idxs, sems = (zbuf0, zbuf1), (idx0, idx1), (sem0, sem1)
        pltpu.sync_copy(zsum_hbm.at[pl.ds(s * _RPT, _RPT)],
                        acc_sh.at[pl.ds(s * _RPT, _RPT)])
        plsc.subcore_barrier()

        def start_load(chunk, slot):
            off = base + chunk * _CH
            pltpu.async_copy(z_hbm.at[pl.ds(off, _CH)], zbufs[slot], sems[slot])
            pltpu.async_copy(y_hbm.at[pl.ds(off, _CH)], idxs[slot], sems[slot])

        def wait_load(chunk, slot):
            off = base + chunk * _CH
            pltpu.make_async_copy(z_hbm.at[pl.ds(off, _CH)], zbufs[slot],
                                  sems[slot]).wait()
            pltpu.make_async_copy(y_hbm.at[pl.ds(off, _CH)], idxs[slot],
                                  sems[slot]).wait()

        def scatter(slot):
            pltpu.sync_copy(zbufs[slot], acc_sh.at[idxs[slot]], add=True)

        start_load(0, 0)

        def body(g, carry):
            i0 = g * 2
            wait_load(i0, 0)
            start_load(i0 + 1, 1)
            scatter(0)
            wait_load(i0 + 1, 1)
            nxt = jnp.where(i0 + 2 < _NITER, i0 + 2, 0)
            start_load(nxt, 0)
            scatter(1)
            return carry

        lax.fori_loop(0, _NPAIR, body, 0)
        wait_load(0, 0)
        plsc.subcore_barrier()
        pltpu.sync_copy(acc_sh.at[pl.ds(s * _RPT, _RPT)],
                        osum_hbm.at[c, pl.ds(s * _RPT, _RPT)])

    return k(z, y, zsum)


def _hist_body(y_ref, out_ref):
    blk = y_ref[0]
    hi_all = blk >> 5
    lo_all = blk & 31
    iot = lax.broadcasted_iota(jnp.int32, (32, _D), 0)
    acc = jnp.zeros((32, 32), jnp.float32)
    for s in range(_HSUB):
        eh = (iot == hi_all[s:s + 1, :]).astype(jnp.float32)
        el = (iot == lo_all[s:s + 1, :]).astype(jnp.float32)
        acc = acc + lax.dot_general(eh, el.T, (((1,), (0,)), ((), ())),
                                    preferred_element_type=jnp.float32)

    @pl.when(pl.program_id(0) == 0)
    def _():
        out_ref[...] = jnp.zeros_like(out_ref)

    out_ref[...] += acc


def _class_counts(y):
    pad = jnp.full((_HP - _N,), _C, jnp.int32)
    y2 = jnp.concatenate([y, pad]).reshape(_HROWS // _HSUB, _HSUB, _D)
    return pl.pallas_call(
        _hist_body,
        grid=(_HROWS // _HSUB,),
        in_specs=[pl.BlockSpec((1, _HSUB, _D), lambda i: (i, 0, 0))],
        out_specs=pl.BlockSpec((32, 32), lambda i: (0, 0)),
        out_shape=jax.ShapeDtypeStruct((32, 32), jnp.float32),
    )(y2)


def _combine_body(psum_ref, cnt_ref, cen_ref, init_ref, oc_ref, oi_ref):
    sums = (psum_ref[0] + psum_ref[1])[:_C]
    counts = cnt_ref[...][:_C]
    centers = cen_ref[...]
    init = init_ref[...]
    present = counts > 0.0
    safe = jnp.where(present, counts, 1.0)
    means = sums / safe
    ema = (1.0 - _MOM) * centers + _MOM * means
    upd = jnp.where(init > 0.0, ema, means)
    oc_ref[...] = jnp.where(present, upd, centers)
    oi_ref[...] = jnp.where(present, 1.0, init)


def kernel(z, y, centers, initialized):
    y = y.astype(jnp.int32)
    zsum = jnp.zeros((_CPAD, _D), jnp.float32)
    counts = _class_counts(y).reshape(_CPAD, 1)
    psum = _sc_segment_sums(z, y, zsum)

    init_f = initialized.astype(jnp.float32).reshape(_C, 1)
    new_centers, new_init = pl.pallas_call(
        _combine_body,
        out_shape=(
            jax.ShapeDtypeStruct((_C, _D), jnp.float32),
            jax.ShapeDtypeStruct((_C, 1), jnp.float32),
        ),
    )(psum, counts, centers, init_f)
    return new_centers, new_init.reshape(_C) > 0.0


# speedup vs baseline: 1.7879x; 1.0546x over previous
import functools

import jax
import jax.numpy as jnp
from jax import lax
from jax.experimental import pallas as pl
from jax.experimental.pallas import tpu as pltpu
from jax.experimental.pallas import tpu_sc as plsc

_N = 320000
_D = 128
_C = 1000
_CPAD = 1024
_MOM = 0.1

_NC = 2
_NS = 16
_NW = _NC * _NS
_ROWS_PER = _N // _NW
_CH = 400
_NITER = _ROWS_PER // _CH
_NPAIR = _NITER // 2
_RPT = _CPAD // _NS

_HROWS = 2560
_HP = _HROWS * _D
_HSUB = 128


def _sc_segment_sums(z, y, zsum):
    mesh = plsc.VectorSubcoreMesh(
        core_axis_name="c", subcore_axis_name="s",
        num_cores=_NC, num_subcores=_NS)

    @functools.partial(
        pl.kernel,
        out_type=jax.ShapeDtypeStruct((_NC, _CPAD, _D), jnp.float32),
        mesh=mesh,
        scratch_types=[
            pltpu.VMEM((_CH, _D), jnp.float32),
            pltpu.VMEM((_CH, _D), jnp.float32),
            pltpu.VMEM((_CH,), jnp.int32),
            pltpu.VMEM((_CH,), jnp.int32),
            pltpu.VMEM_SHARED((_CPAD, _D), jnp.float32),
            pltpu.SemaphoreType.DMA,
            pltpu.SemaphoreType.DMA,
        ],
    )
    def k(z_hbm, y_hbm, zsum_hbm, osum_hbm,
          zbuf0, zbuf1, idx0, idx1, acc_sh, sem0, sem1):
        c = lax.axis_index("c")
        s = lax.axis_index("s")
        wid = c * _NS + s
        base = wid * _ROWS_PER
        zbufs, idxs, sems = (zbuf0, zbuf1), (idx0, idx1), (sem0, sem1)
        pltpu.sync_copy(zsum_hbm.at[pl.ds(s * _RPT, _RPT)],
                        acc_sh.at[pl.ds(s * _RPT, _RPT)])
        plsc.subcore_barrier()

        def start_load(chunk, slot):
            off = base + chunk * _CH
            pltpu.async_copy(z_hbm.at[pl.ds(off, _CH)], zbufs[slot], sems[slot])
            pltpu.async_copy(y_hbm.at[pl.ds(off, _CH)], idxs[slot], sems[slot])

        def wait_load(chunk, slot):
            off = base + chunk * _CH
            pltpu.make_async_copy(z_hbm.at[pl.ds(off, _CH)], zbufs[slot],
                                  sems[slot]).wait()
            pltpu.make_async_copy(y_hbm.at[pl.ds(off, _CH)], idxs[slot],
                                  sems[slot]).wait()

        def scatter(slot):
            pltpu.sync_copy(zbufs[slot], acc_sh.at[idxs[slot]], add=True)

        start_load(0, 0)

        def body(g, carry):
            i0 = g * 2
            wait_load(i0, 0)
            start_load(i0 + 1, 1)
            scatter(0)
            wait_load(i0 + 1, 1)
            start_load(i0 + 2, 0)
            scatter(1)
            return carry

        lax.fori_loop(0, _NPAIR, body, 0)
        wait_load(_NITER - 1, 0)
        scatter(0)
        plsc.subcore_barrier()
        pltpu.sync_copy(acc_sh.at[pl.ds(s * _RPT, _RPT)],
                        osum_hbm.at[c, pl.ds(s * _RPT, _RPT)])

    return k(z, y, zsum)


def _hist_body(y_ref, out_ref):
    blk = y_ref[0]
    hi_all = blk >> 5
    lo_all = blk & 31
    iot = lax.broadcasted_iota(jnp.int32, (32, _D), 0)
    acc = jnp.zeros((32, 32), jnp.float32)
    for s in range(_HSUB):
        eh = (iot == hi_all[s:s + 1, :]).astype(jnp.float32)
        el = (iot == lo_all[s:s + 1, :]).astype(jnp.float32)
        acc = acc + lax.dot_general(eh, el.T, (((1,), (0,)), ((), ())),
                                    preferred_element_type=jnp.float32)

    @pl.when(pl.program_id(0) == 0)
    def _():
        out_ref[...] = jnp.zeros_like(out_ref)

    out_ref[...] += acc


def _class_counts(y):
    pad = jnp.full((_HP - _N,), _C, jnp.int32)
    y2 = jnp.concatenate([y, pad]).reshape(_HROWS // _HSUB, _HSUB, _D)
    return pl.pallas_call(
        _hist_body,
        grid=(_HROWS // _HSUB,),
        in_specs=[pl.BlockSpec((1, _HSUB, _D), lambda i: (i, 0, 0))],
        out_specs=pl.BlockSpec((32, 32), lambda i: (0, 0)),
        out_shape=jax.ShapeDtypeStruct((32, 32), jnp.float32),
    )(y2)


def _combine_body(psum_ref, cnt_ref, cen_ref, init_ref, oc_ref, oi_ref):
    sums = (psum_ref[0] + psum_ref[1])[:_C]
    counts = cnt_ref[...][:_C]
    centers = cen_ref[...]
    init = init_ref[...]
    present = counts > 0.0
    safe = jnp.where(present, counts, 1.0)
    means = sums / safe
    ema = (1.0 - _MOM) * centers + _MOM * means
    upd = jnp.where(init > 0.0, ema, means)
    oc_ref[...] = jnp.where(present, upd, centers)
    oi_ref[...] = jnp.where(present, 1.0, init)


def kernel(z, y, centers, initialized):
    y = y.astype(jnp.int32)
    zsum = jnp.zeros((_CPAD, _D), jnp.float32)
    counts = _class_counts(y).reshape(_CPAD, 1)
    psum = _sc_segment_sums(z, y, zsum)

    init_f = initialized.astype(jnp.float32).reshape(_C, 1)
    new_centers, new_init = pl.pallas_call(
        _combine_body,
        out_shape=(
            jax.ShapeDtypeStruct((_C, _D), jnp.float32),
            jax.ShapeDtypeStruct((_C, 1), jnp.float32),
        ),
    )(psum, counts, centers, init_f)
    return new_centers, new_init.reshape(_C) > 0.0
